# baseline (speedup 1.0000x reference)
import functools

import jax
import jax.numpy as jnp
from jax import lax
from jax.experimental import pallas as pl
from jax.experimental.pallas import tpu as pltpu
from jax.experimental.pallas import tpu_sc as plsc

_NC = 2
_NS = 16
_NW = _NC * _NS
_CH = 80
_NPAD = 10240

_EPS = 1e-5


def _gn(x, g, b):
    mu = jnp.mean(x, axis=1, keepdims=True)
    var = jnp.mean((x - mu) ** 2, axis=1, keepdims=True)
    return (x - mu) / jnp.sqrt(var + _EPS) * g + b


def _pre_body(agts_r, ctx_r, wq_r, gq_r, beq_r, wc0q_r, wc0x_r, wa_r,
              qc_r, xc_r, base_r):
    agts = agts_r[...]
    q = jax.nn.relu(_gn(jnp.dot(agts, wq_r[...], preferred_element_type=jnp.float32),
                        gq_r[...], beq_r[...]))
    qc_r[...] = jnp.dot(q, wc0q_r[...], preferred_element_type=jnp.float32)
    xc_r[...] = jnp.dot(ctx_r[...], wc0x_r[...], preferred_element_type=jnp.float32)
    base_r[...] = jnp.dot(agts, wa_r[...], preferred_element_type=jnp.float32)


def _build_tables(agts, ctx, W_q, g_q, be_q, Wc0_q, Wc0_x, W_a):
    n, d = agts.shape
    blk = 1000
    grid = n // blk
    full = lambda r, c: pl.BlockSpec((r, c), lambda i: (0, 0))
    return pl.pallas_call(
        _pre_body,
        grid=(grid,),
        in_specs=[
            pl.BlockSpec((blk, d), lambda i: (i, 0)),
            pl.BlockSpec((blk, d), lambda i: (i, 0)),
            full(d, d), full(1, d), full(1, d),
            full(d, d), full(d, d), full(d, d),
        ],
        out_specs=[
            pl.BlockSpec((blk, d), lambda i: (i, 0)),
            pl.BlockSpec((blk, d), lambda i: (i, 0)),
            pl.BlockSpec((blk, d), lambda i: (i, 0)),
        ],
        out_shape=[
            jax.ShapeDtypeStruct((n, d), jnp.float32),
            jax.ShapeDtypeStruct((n, d), jnp.float32),
            jax.ShapeDtypeStruct((n, d), jnp.float32),
        ],
    )(agts, ctx, W_q, g_q, be_q, Wc0_q, Wc0_x, W_a)


def _gather_rows(qc_tab, xc_tab, ax_t, ay_t, cx_t, cy_t, hi, wi):
    e = hi.shape[0]
    n, d = qc_tab.shape
    per_w = e // _NW
    nch = per_w // _CH
    mesh = plsc.VectorSubcoreMesh(core_axis_name="c", subcore_axis_name="s",
                                  num_cores=_NC, num_subcores=_NS)

    @functools.partial(
        pl.kernel,
        out_type=(jax.ShapeDtypeStruct((e, d), jnp.float32),
                  jax.ShapeDtypeStruct((e, d), jnp.float32),
                  jax.ShapeDtypeStruct((e,), jnp.float32),
                  jax.ShapeDtypeStruct((e,), jnp.float32),
                  jax.ShapeDtypeStruct((e,), jnp.float32),
                  jax.ShapeDtypeStruct((e,), jnp.float32)),
        mesh=mesh,
        scratch_types=[
            pltpu.VMEM((per_w,), jnp.int32),
            pltpu.VMEM((per_w,), jnp.int32),
            pltpu.VMEM((per_w,), jnp.float32),
            pltpu.VMEM((per_w,), jnp.float32),
            pltpu.VMEM((per_w,), jnp.float32),
            pltpu.VMEM((per_w,), jnp.float32),
            pltpu.VMEM((_CH, 128), jnp.float32),
            pltpu.VMEM((_CH, 128), jnp.float32),
            pltpu.SemaphoreType.DMA,
            pltpu.SemaphoreType.DMA,
            pltpu.SemaphoreType.DMA,
        ],
    )
    def gather_k(qc_hbm, xc_hbm, ax_hbm, ay_hbm, cx_hbm, cy_hbm,
                 hi_hbm, wi_hbm,
                 g1_hbm, g2_hbm, ax_o, ay_o, cx_o, cy_o,
                 hi_v, wi_v, axb, ayb, cxb, cyb, r1, r2, s1, s2, s3):
        wid = lax.axis_index("s") * _NC + lax.axis_index("c")
        e0 = wid * per_w
        pltpu.sync_copy(hi_hbm.at[pl.ds(e0, per_w)], hi_v)
        pltpu.sync_copy(wi_hbm.at[pl.ds(e0, per_w)], wi_v)

        cpa = pltpu.async_copy(ax_hbm.at[hi_v], axb, s3)
        cpb = pltpu.async_copy(ay_hbm.at[hi_v], ayb, s3)
        cpc = pltpu.async_copy(cx_hbm.at[wi_v], cxb, s3)
        cpd = pltpu.async_copy(cy_hbm.at[wi_v], cyb, s3)

        def body(j, carry):
            base = e0 + j * _CH
            cp1 = pltpu.async_copy(qc_hbm.at[hi_v.at[pl.ds(j * _CH, _CH)]], r1, s1)
            cp2 = pltpu.async_copy(xc_hbm.at[wi_v.at[pl.ds(j * _CH, _CH)]], r2, s2)
            cp1.wait()
            cp2.wait()
            pltpu.sync_copy(r1, g1_hbm.at[pl.ds(base, _CH)])
            pltpu.sync_copy(r2, g2_hbm.at[pl.ds(base, _CH)])
            return carry

        lax.fori_loop(0, nch, body, 0)

        cpa.wait()
        cpb.wait()
        cpc.wait()
        cpd.wait()
        pltpu.sync_copy(axb, ax_o.at[pl.ds(e0, per_w)])
        pltpu.sync_copy(ayb, ay_o.at[pl.ds(e0, per_w)])
        pltpu.sync_copy(cxb, cx_o.at[pl.ds(e0, per_w)])
        pltpu.sync_copy(cyb, cy_o.at[pl.ds(e0, per_w)])

    return gather_k(qc_tab, xc_tab, ax_t, ay_t, cx_t, cy_t, hi, wi)


def _edge_body(g1_r, g2_r, ax_r, ay_r, cx_r, cy_r, wd0_r, bd0_r, wd1_r,
               gd_r, bed_r, wc0d_r, gc0_r, bec0_r, h_r):
    wd0 = wd0_r[...]
    d1 = jax.nn.relu((ax_r[...] - cx_r[...]) * wd0[0:1, :]
                     + (ay_r[...] - cy_r[...]) * wd0[1:2, :]
                     + bd0_r[...])
    d2 = jax.nn.relu(_gn(jnp.dot(d1, wd1_r[...], preferred_element_type=jnp.float32),
                         gd_r[...], bed_r[...]))
    pre = (jnp.dot(d2, wc0d_r[...], preferred_element_type=jnp.float32)
           + g1_r[...] + g2_r[...])
    h_r[...] = jax.nn.relu(_gn(pre, gc0_r[...], bec0_r[...]))


def _edge_mlp(g1, g2, ax, ay, cx, cy, W_d0, b_d0, W_d1, g_d, be_d,
              Wc0_d, g_c0, be_c0):
    e, d = g1.shape
    blk = 2000
    grid = e // blk
    full = lambda r, c: pl.BlockSpec((r, c), lambda i: (0, 0))
    col = pl.BlockSpec((blk, 1), lambda i: (i, 0))
    return pl.pallas_call(
        _edge_body,
        grid=(grid,),
        in_specs=[
            pl.BlockSpec((blk, d), lambda i: (i, 0)),
            pl.BlockSpec((blk, d), lambda i: (i, 0)),
            col, col, col, col,
            full(2, d), full(1, d),
            full(d, d), full(1, d), full(1, d),
            full(d, d), full(1, d), full(1, d),
        ],
        out_specs=pl.BlockSpec((blk, d), lambda i: (i, 0)),
        out_shape=jax.ShapeDtypeStruct((e, d), jnp.float32),
    )(g1, g2, ax.reshape(e, 1), ay.reshape(e, 1), cx.reshape(e, 1),
      cy.reshape(e, 1), W_d0, b_d0, W_d1, g_d, be_d, Wc0_d, g_c0, be_c0)


def _scatter_add(h, hi):
    e, d = h.shape
    per_w = e // _NW
    nch = per_w // _CH
    stripe = _NPAD // _NS
    zrows = stripe // 5
    mesh = plsc.VectorSubcoreMesh(core_axis_name="c", subcore_axis_name="s",
                                  num_cores=_NC, num_subcores=_NS)

    @functools.partial(
        pl.kernel,
        out_type=jax.ShapeDtypeStruct((_NC, _NPAD, d), jnp.float32),
        mesh=mesh,
        scratch_types=[
            pltpu.VMEM((per_w,), jnp.int32),
            pltpu.VMEM((_CH, d), jnp.float32),
            pltpu.VMEM((zrows, d), jnp.float32),
            pltpu.VMEM_SHARED((_NPAD, d), jnp.float32),
        ],
    )
    def scatter_k(h_hbm, hi_hbm, s_out, hi_v, hbuf, zbuf, s_sh):
        cid = lax.axis_index("c")
        sid = lax.axis_index("s")
        wid = sid * _NC + cid
        e0 = wid * per_w
        pltpu.sync_copy(hi_hbm.at[pl.ds(e0, per_w)], hi_v)

        def zb(i, carry):
            zbuf[i // 8, pl.ds((i % 8) * 16, 16)] = jnp.zeros((16,), jnp.float32)
            return carry

        lax.fori_loop(0, zrows * 8, zb, 0)

        def zc(p, carry):
            pltpu.sync_copy(zbuf, s_sh.at[pl.ds(sid * stripe + p * zrows, zrows)])
            return carry

        lax.fori_loop(0, 5, zc, 0)
        plsc.subcore_barrier()

        def body(j, carry):
            base = e0 + j * _CH
            pltpu.sync_copy(h_hbm.at[pl.ds(base, _CH)], hbuf)
            pltpu.sync_copy(hbuf, s_sh.at[hi_v.at[pl.ds(j * _CH, _CH)]], add=True)
            return carry

        lax.fori_loop(0, nch, body, 0)
        plsc.subcore_barrier()
        pltpu.sync_copy(s_sh.at[pl.ds(sid * stripe, stripe)],
                        s_out.at[cid, pl.ds(sid * stripe, stripe)])

    return scatter_k(h, hi)


def _final_body(s_r, base_r, agts_r, wc1_r, gn_r, ben_r, wl_r, gl_r, bel_r, o_r):
    s = s_r[0] + s_r[1]
    out = base_r[...] + jnp.dot(s, wc1_r[...], preferred_element_type=jnp.float32)
    out = jax.nn.relu(_gn(out, gn_r[...], ben_r[...]))
    out = _gn(jnp.dot(out, wl_r[...], preferred_element_type=jnp.float32),
              gl_r[...], bel_r[...])
    o_r[...] = jax.nn.relu(out + agts_r[...])


def _final(s_parts, base, agts, W_c1, g_n, be_n, W_l, g_l, be_l):
    n, d = agts.shape
    blk = 1000
    grid = n // blk
    full = lambda r, c: pl.BlockSpec((r, c), lambda i: (0, 0))
    return pl.pallas_call(
        _final_body,
        grid=(grid,),
        in_specs=[
            pl.BlockSpec((_NC, blk, d), lambda i: (0, i, 0)),
            pl.BlockSpec((blk, d), lambda i: (i, 0)),
            pl.BlockSpec((blk, d), lambda i: (i, 0)),
            full(d, d), full(1, d), full(1, d),
            full(d, d), full(1, d), full(1, d),
        ],
        out_specs=pl.BlockSpec((blk, d), lambda i: (i, 0)),
        out_shape=jax.ShapeDtypeStruct((n, d), jnp.float32),
    )(s_parts, base, agts, W_c1, g_n, be_n, W_l, g_l, be_l)


def kernel(agts, ctx, agt_ctrs, ctx_ctrs, W_d0, b_d0, W_d1, g_d, be_d,
           W_q, g_q, be_q, W_c0, g_c0, be_c0, W_c1, W_a, g_n, be_n,
           W_l, g_l, be_l, hi, wi):
    n, d = agts.shape
    row = lambda v: v.reshape(1, d)
    Wc0_d, Wc0_q, Wc0_x = W_c0[:d], W_c0[d:2 * d], W_c0[2 * d:]

    qc_tab, xc_tab, base = _build_tables(
        agts, ctx, W_q, row(g_q), row(be_q), Wc0_q, Wc0_x, W_a)

    actr = jnp.asarray(agt_ctrs)
    cctr = jnp.asarray(ctx_ctrs)
    g1, g2, ax, ay, cx, cy = _gather_rows(
        qc_tab, xc_tab, actr[:, 0], actr[:, 1], cctr[:, 0], cctr[:, 1],
        hi, wi)

    h = _edge_mlp(g1, g2, ax, ay, cx, cy, W_d0, row(b_d0), W_d1, row(g_d),
                  row(be_d), Wc0_d, row(g_c0), row(be_c0))

    s_parts = _scatter_add(h, hi)[:, :n, :]

    return _final(s_parts, base, agts, W_c1, row(g_n), row(be_n), W_l,
                  row(g_l), row(be_l))


# speedup vs baseline: 1.0478x; 1.0478x over previous
import functools

import jax
import jax.numpy as jnp
from jax import lax
from jax.experimental import pallas as pl
from jax.experimental.pallas import tpu as pltpu
from jax.experimental.pallas import tpu_sc as plsc

_NC = 2
_NS = 16
_NW = _NC * _NS
_CH = 40
_KB = 5
_NPAD = 10240

_EPS = 1e-5


def _gn(x, g, b):
    mu = jnp.mean(x, axis=1, keepdims=True)
    var = jnp.mean((x - mu) ** 2, axis=1, keepdims=True)
    return (x - mu) / jnp.sqrt(var + _EPS) * g + b


def _pre_body(agts_r, ctx_r, wq_r, gq_r, beq_r, wc0q_r, wc0x_r, wa_r,
              qc_r, xc_r, base_r):
    agts = agts_r[...]
    q = jax.nn.relu(_gn(jnp.dot(agts, wq_r[...], preferred_element_type=jnp.float32),
                        gq_r[...], beq_r[...]))
    qc_r[...] = jnp.dot(q, wc0q_r[...], preferred_element_type=jnp.float32)
    xc_r[...] = jnp.dot(ctx_r[...], wc0x_r[...], preferred_element_type=jnp.float32)
    base_r[...] = jnp.dot(agts, wa_r[...], preferred_element_type=jnp.float32)


def _build_tables(agts, ctx, W_q, g_q, be_q, Wc0_q, Wc0_x, W_a):
    n, d = agts.shape
    blk = 1000
    grid = n // blk
    full = lambda r, c: pl.BlockSpec((r, c), lambda i: (0, 0))
    return pl.pallas_call(
        _pre_body,
        grid=(grid,),
        in_specs=[
            pl.BlockSpec((blk, d), lambda i: (i, 0)),
            pl.BlockSpec((blk, d), lambda i: (i, 0)),
            full(d, d), full(1, d), full(1, d),
            full(d, d), full(d, d), full(d, d),
        ],
        out_specs=[
            pl.BlockSpec((blk, d), lambda i: (i, 0)),
            pl.BlockSpec((blk, d), lambda i: (i, 0)),
            pl.BlockSpec((blk, d), lambda i: (i, 0)),
        ],
        out_shape=[
            jax.ShapeDtypeStruct((n, d), jnp.float32),
            jax.ShapeDtypeStruct((n, d), jnp.float32),
            jax.ShapeDtypeStruct((n, d), jnp.float32),
        ],
    )(agts, ctx, W_q, g_q, be_q, Wc0_q, Wc0_x, W_a)


def _gather_rows(qc_tab, xc_tab, ax_t, ay_t, cx_t, cy_t, hi, wi):
    e = hi.shape[0]
    n, d = qc_tab.shape
    per_w = e // _NW
    nch = per_w // _CH
    nsc = nch // _KB
    mesh = plsc.VectorSubcoreMesh(core_axis_name="c", subcore_axis_name="s",
                                  num_cores=_NC, num_subcores=_NS)

    @functools.partial(
        pl.kernel,
        out_type=(jax.ShapeDtypeStruct((e, d), jnp.float32),
                  jax.ShapeDtypeStruct((e, d), jnp.float32),
                  jax.ShapeDtypeStruct((e,), jnp.float32),
                  jax.ShapeDtypeStruct((e,), jnp.float32),
                  jax.ShapeDtypeStruct((e,), jnp.float32),
                  jax.ShapeDtypeStruct((e,), jnp.float32)),
        mesh=mesh,
        scratch_types=(
            [pltpu.VMEM((per_w,), jnp.int32)] * 2
            + [pltpu.VMEM((per_w,), jnp.float32)] * 4
            + [pltpu.VMEM((_CH, 128), jnp.float32)] * (2 * _KB)
            + [pltpu.SemaphoreType.DMA] * (2 * _KB + 1)
        ),
    )
    def gather_k(qc_hbm, xc_hbm, ax_hbm, ay_hbm, cx_hbm, cy_hbm,
                 hi_hbm, wi_hbm,
                 g1_hbm, g2_hbm, ax_o, ay_o, cx_o, cy_o,
                 hi_v, wi_v, axb, ayb, cxb, cyb, *bufs_and_sems):
        r1 = bufs_and_sems[:_KB]
        r2 = bufs_and_sems[_KB:2 * _KB]
        sg = bufs_and_sems[2 * _KB:3 * _KB]
        sw = bufs_and_sems[3 * _KB:4 * _KB]
        s3 = bufs_and_sems[4 * _KB]
        wid = lax.axis_index("s") * _NC + lax.axis_index("c")
        e0 = wid * per_w
        pltpu.sync_copy(hi_hbm.at[pl.ds(e0, per_w)], hi_v)
        pltpu.sync_copy(wi_hbm.at[pl.ds(e0, per_w)], wi_v)

        cpa = pltpu.async_copy(ax_hbm.at[hi_v], axb, s3)
        cpb = pltpu.async_copy(ay_hbm.at[hi_v], ayb, s3)
        cpc = pltpu.async_copy(cx_hbm.at[wi_v], cxb, s3)
        cpd = pltpu.async_copy(cy_hbm.at[wi_v], cyb, s3)

        def body(s, carry):
            gets = []
            for b in range(_KB):
                j = s * _KB + b
                ids = pl.ds(j * _CH, _CH)
                gets.append((
                    pltpu.async_copy(qc_hbm.at[hi_v.at[ids]], r1[b], sg[b]),
                    pltpu.async_copy(xc_hbm.at[wi_v.at[ids]], r2[b], sg[b]),
                ))
            puts = []
            for b in range(_KB):
                j = s * _KB + b
                base = e0 + j * _CH
                gets[b][0].wait()
                gets[b][1].wait()
                puts.append((
                    pltpu.async_copy(r1[b], g1_hbm.at[pl.ds(base, _CH)], sw[b]),
                    pltpu.async_copy(r2[b], g2_hbm.at[pl.ds(base, _CH)], sw[b]),
                ))
            for b in range(_KB):
                puts[b][0].wait()
                puts[b][1].wait()
            return carry

        lax.fori_loop(0, nsc, body, 0)

        cpa.wait()
        cpb.wait()
        cpc.wait()
        cpd.wait()
        pltpu.sync_copy(axb, ax_o.at[pl.ds(e0, per_w)])
        pltpu.sync_copy(ayb, ay_o.at[pl.ds(e0, per_w)])
        pltpu.sync_copy(cxb, cx_o.at[pl.ds(e0, per_w)])
        pltpu.sync_copy(cyb, cy_o.at[pl.ds(e0, per_w)])

    return gather_k(qc_tab, xc_tab, ax_t, ay_t, cx_t, cy_t, hi, wi)


def _edge_body(g1_r, g2_r, ax_r, ay_r, cx_r, cy_r, wd0_r, bd0_r, wd1_r,
               gd_r, bed_r, wc0d_r, gc0_r, bec0_r, h_r):
    wd0 = wd0_r[...]
    d1 = jax.nn.relu((ax_r[...] - cx_r[...]) * wd0[0:1, :]
                     + (ay_r[...] - cy_r[...]) * wd0[1:2, :]
                     + bd0_r[...])
    d2 = jax.nn.relu(_gn(jnp.dot(d1, wd1_r[...], preferred_element_type=jnp.float32),
                         gd_r[...], bed_r[...]))
    pre = (jnp.dot(d2, wc0d_r[...], preferred_element_type=jnp.float32)
           + g1_r[...] + g2_r[...])
    h_r[...] = jax.nn.relu(_gn(pre, gc0_r[...], bec0_r[...]))


def _edge_mlp(g1, g2, ax, ay, cx, cy, W_d0, b_d0, W_d1, g_d, be_d,
              Wc0_d, g_c0, be_c0):
    e, d = g1.shape
    blk = 2000
    grid = e // blk
    full = lambda r, c: pl.BlockSpec((r, c), lambda i: (0, 0))
    col = pl.BlockSpec((blk, 1), lambda i: (i, 0))
    return pl.pallas_call(
        _edge_body,
        grid=(grid,),
        in_specs=[
            pl.BlockSpec((blk, d), lambda i: (i, 0)),
            pl.BlockSpec((blk, d), lambda i: (i, 0)),
            col, col, col, col,
            full(2, d), full(1, d),
            full(d, d), full(1, d), full(1, d),
            full(d, d), full(1, d), full(1, d),
        ],
        out_specs=pl.BlockSpec((blk, d), lambda i: (i, 0)),
        out_shape=jax.ShapeDtypeStruct((e, d), jnp.float32),
    )(g1, g2, ax.reshape(e, 1), ay.reshape(e, 1), cx.reshape(e, 1),
      cy.reshape(e, 1), W_d0, b_d0, W_d1, g_d, be_d, Wc0_d, g_c0, be_c0)


_SKB = 2


def _scatter_add(h, hi):
    e, d = h.shape
    per_w = e // _NW
    nch = per_w // _CH
    nsc = nch // _SKB
    stripe = _NPAD // _NS
    zrows = stripe // 10
    mesh = plsc.VectorSubcoreMesh(core_axis_name="c", subcore_axis_name="s",
                                  num_cores=_NC, num_subcores=_NS)

    @functools.partial(
        pl.kernel,
        out_type=jax.ShapeDtypeStruct((_NC, _NPAD, d), jnp.float32),
        mesh=mesh,
        scratch_types=(
            [pltpu.VMEM((per_w,), jnp.int32)]
            + [pltpu.VMEM((_CH, d), jnp.float32)] * _SKB
            + [pltpu.VMEM((zrows, d), jnp.float32)]
            + [pltpu.VMEM_SHARED((_NPAD, d), jnp.float32)]
            + [pltpu.SemaphoreType.DMA] * (2 * _SKB)
        ),
    )
    def scatter_k(h_hbm, hi_hbm, s_out, hi_v, *rest):
        hbuf = rest[:_SKB]
        zbuf = rest[_SKB]
        s_sh = rest[_SKB + 1]
        sr = rest[_SKB + 2:2 * _SKB + 2]
        sa = rest[2 * _SKB + 2:3 * _SKB + 2]
        cid = lax.axis_index("c")
        sid = lax.axis_index("s")
        wid = sid * _NC + cid
        e0 = wid * per_w
        pltpu.sync_copy(hi_hbm.at[pl.ds(e0, per_w)], hi_v)

        def zb(i, carry):
            zbuf[i // 8, pl.ds((i % 8) * 16, 16)] = jnp.zeros((16,), jnp.float32)
            return carry

        lax.fori_loop(0, zrows * 8, zb, 0)

        def zc(p, carry):
            pltpu.sync_copy(zbuf, s_sh.at[pl.ds(sid * stripe + p * zrows, zrows)])
            return carry

        lax.fori_loop(0, 10, zc, 0)
        plsc.subcore_barrier()

        def body(s, carry):
            gets = []
            for b in range(_SKB):
                j = s * _SKB + b
                base = e0 + j * _CH
                gets.append(pltpu.async_copy(
                    h_hbm.at[pl.ds(base, _CH)], hbuf[b], sr[b]))
            adds = []
            for b in range(_SKB):
                j = s * _SKB + b
                gets[b].wait()
                adds.append(pltpu.async_copy(
                    hbuf[b], s_sh.at[hi_v.at[pl.ds(j * _CH, _CH)]], sa[b],
                    add=True))
            for b in range(_SKB):
                adds[b].wait()
            return carry

        lax.fori_loop(0, nsc, body, 0)
        plsc.subcore_barrier()
        pltpu.sync_copy(s_sh.at[pl.ds(sid * stripe, stripe)],
                        s_out.at[cid, pl.ds(sid * stripe, stripe)])

    return scatter_k(h, hi)


def _final_body(s_r, base_r, agts_r, wc1_r, gn_r, ben_r, wl_r, gl_r, bel_r, o_r):
    s = s_r[0] + s_r[1]
    out = base_r[...] + jnp.dot(s, wc1_r[...], preferred_element_type=jnp.float32)
    out = jax.nn.relu(_gn(out, gn_r[...], ben_r[...]))
    out = _gn(jnp.dot(out, wl_r[...], preferred_element_type=jnp.float32),
              gl_r[...], bel_r[...])
    o_r[...] = jax.nn.relu(out + agts_r[...])


def _final(s_parts, base, agts, W_c1, g_n, be_n, W_l, g_l, be_l):
    n, d = agts.shape
    blk = 1000
    grid = n // blk
    full = lambda r, c: pl.BlockSpec((r, c), lambda i: (0, 0))
    return pl.pallas_call(
        _final_body,
        grid=(grid,),
        in_specs=[
            pl.BlockSpec((_NC, blk, d), lambda i: (0, i, 0)),
            pl.BlockSpec((blk, d), lambda i: (i, 0)),
            pl.BlockSpec((blk, d), lambda i: (i, 0)),
            full(d, d), full(1, d), full(1, d),
            full(d, d), full(1, d), full(1, d),
        ],
        out_specs=pl.BlockSpec((blk, d), lambda i: (i, 0)),
        out_shape=jax.ShapeDtypeStruct((n, d), jnp.float32),
    )(s_parts, base, agts, W_c1, g_n, be_n, W_l, g_l, be_l)


def kernel(agts, ctx, agt_ctrs, ctx_ctrs, W_d0, b_d0, W_d1, g_d, be_d,
           W_q, g_q, be_q, W_c0, g_c0, be_c0, W_c1, W_a, g_n, be_n,
           W_l, g_l, be_l, hi, wi):
    n, d = agts.shape
    row = lambda v: v.reshape(1, d)
    Wc0_d, Wc0_q, Wc0_x = W_c0[:d], W_c0[d:2 * d], W_c0[2 * d:]

    qc_tab, xc_tab, base = _build_tables(
        agts, ctx, W_q, row(g_q), row(be_q), Wc0_q, Wc0_x, W_a)

    actr = jnp.asarray(agt_ctrs)
    cctr = jnp.asarray(ctx_ctrs)
    g1, g2, ax, ay, cx, cy = _gather_rows(
        qc_tab, xc_tab, actr[:, 0], actr[:, 1], cctr[:, 0], cctr[:, 1],
        hi, wi)

    h = _edge_mlp(g1, g2, ax, ay, cx, cy, W_d0, row(b_d0), W_d1, row(g_d),
                  row(be_d), Wc0_d, row(g_c0), row(be_c0))

    s_parts = _scatter_add(h, hi)[:, :n, :]

    return _final(s_parts, base, agts, W_c1, row(g_n), row(be_n), W_l,
                  row(g_l), row(be_l))


# speedup vs baseline: 1.4469x; 1.3809x over previous
import functools

import jax
import jax.numpy as jnp
from jax import lax
from jax.experimental import pallas as pl
from jax.experimental.pallas import tpu as pltpu
from jax.experimental.pallas import tpu_sc as plsc

_NC = 2
_NS = 16
_NW = _NC * _NS
_CH = 40
_KB = 5
_NPAD = 10240

_EPS = 1e-5


def _gn(x, g, b):
    mu = jnp.mean(x, axis=1, keepdims=True)
    var = jnp.mean((x - mu) ** 2, axis=1, keepdims=True)
    return (x - mu) / jnp.sqrt(var + _EPS) * g + b


def _pre_body(agts_r, ctx_r, wq_r, gq_r, beq_r, wc0q_r, wc0x_r, wa_r,
              qc_r, xc_r, base_r):
    agts = agts_r[...]
    q = jax.nn.relu(_gn(jnp.dot(agts, wq_r[...], preferred_element_type=jnp.float32),
                        gq_r[...], beq_r[...]))
    qc_r[...] = jnp.dot(q, wc0q_r[...], preferred_element_type=jnp.float32)
    xc_r[...] = jnp.dot(ctx_r[...], wc0x_r[...], preferred_element_type=jnp.float32)
    base_r[...] = jnp.dot(agts, wa_r[...], preferred_element_type=jnp.float32)


def _build_tables(agts, ctx, W_q, g_q, be_q, Wc0_q, Wc0_x, W_a):
    n, d = agts.shape
    blk = 1000
    grid = n // blk
    full = lambda r, c: pl.BlockSpec((r, c), lambda i: (0, 0))
    return pl.pallas_call(
        _pre_body,
        grid=(grid,),
        in_specs=[
            pl.BlockSpec((blk, d), lambda i: (i, 0)),
            pl.BlockSpec((blk, d), lambda i: (i, 0)),
            full(d, d), full(1, d), full(1, d),
            full(d, d), full(d, d), full(d, d),
        ],
        out_specs=[
            pl.BlockSpec((blk, d), lambda i: (i, 0)),
            pl.BlockSpec((blk, d), lambda i: (i, 0)),
            pl.BlockSpec((blk, d), lambda i: (i, 0)),
        ],
        out_shape=[
            jax.ShapeDtypeStruct((n, d), jnp.float32),
            jax.ShapeDtypeStruct((n, d), jnp.float32),
            jax.ShapeDtypeStruct((n, d), jnp.float32),
        ],
    )(agts, ctx, W_q, g_q, be_q, Wc0_q, Wc0_x, W_a)


def _gather_rows(qc_tab, xc_tab, actr_f, cctr_f, hi, wi):
    e = hi.shape[0]
    n, d = qc_tab.shape
    n2 = actr_f.shape[0]
    per_w = e // _NW
    nch = per_w // _CH
    nsc = nch // _KB
    nv = per_w // 16
    mesh = plsc.VectorSubcoreMesh(core_axis_name="c", subcore_axis_name="s",
                                  num_cores=_NC, num_subcores=_NS)

    @functools.partial(
        pl.kernel,
        out_type=(jax.ShapeDtypeStruct((e, d), jnp.float32),
                  jax.ShapeDtypeStruct((e, d), jnp.float32),
                  jax.ShapeDtypeStruct((e,), jnp.float32),
                  jax.ShapeDtypeStruct((e,), jnp.float32)),
        mesh=mesh,
        compiler_params=pltpu.CompilerParams(needs_layout_passes=False),
        scratch_types=(
            [pltpu.VMEM((per_w,), jnp.int32)] * 2
            + [pltpu.VMEM((n2,), jnp.float32)] * 2
            + [pltpu.VMEM((per_w,), jnp.float32)]
            + [pltpu.VMEM((_CH, 128), jnp.float32)] * (2 * _KB)
            + [pltpu.SemaphoreType.DMA] * (2 * _KB + 1)
        ),
    )
    def gather_k(qc_hbm, xc_hbm, actr_hbm, cctr_hbm,
                 hi_hbm, wi_hbm,
                 g1_hbm, g2_hbm, dx_o, dy_o,
                 hi_v, wi_v, actr_v, cctr_v, dbuf, *bufs_and_sems):
        r1 = bufs_and_sems[:_KB]
        r2 = bufs_and_sems[_KB:2 * _KB]
        sg = bufs_and_sems[2 * _KB:3 * _KB]
        sw = bufs_and_sems[3 * _KB:4 * _KB]
        s3 = bufs_and_sems[4 * _KB]
        wid = lax.axis_index("s") * _NC + lax.axis_index("c")
        e0 = wid * per_w
        pltpu.sync_copy(hi_hbm.at[pl.ds(e0, per_w)], hi_v)
        pltpu.sync_copy(wi_hbm.at[pl.ds(e0, per_w)], wi_v)
        pltpu.sync_copy(actr_hbm, actr_v)
        pltpu.sync_copy(cctr_hbm, cctr_v)

        def diff(off, out_ref):
            def dbody(i, c):
                sl = pl.ds(i * 16, 16)
                a = plsc.load_gather(actr_v, [hi_v[sl] * 2 + off])
                cc = plsc.load_gather(cctr_v, [wi_v[sl] * 2 + off])
                dbuf[sl] = a - cc
                return c

            lax.fori_loop(0, nv, dbody, 0)
            pltpu.sync_copy(dbuf, out_ref.at[pl.ds(e0, per_w)])

        diff(0, dx_o)
        diff(1, dy_o)

        def body(s, carry):
            gets = []
            for b in range(_KB):
                j = s * _KB + b
                ids = pl.ds(j * _CH, _CH)
                gets.append((
                    pltpu.async_copy(qc_hbm.at[hi_v.at[ids]], r1[b], sg[b]),
                    pltpu.async_copy(xc_hbm.at[wi_v.at[ids]], r2[b], sg[b]),
                ))
            puts = []
            for b in range(_KB):
                j = s * _KB + b
                base = e0 + j * _CH
                gets[b][0].wait()
                gets[b][1].wait()
                puts.append((
                    pltpu.async_copy(r1[b], g1_hbm.at[pl.ds(base, _CH)], sw[b]),
                    pltpu.async_copy(r2[b], g2_hbm.at[pl.ds(base, _CH)], sw[b]),
                ))
            for b in range(_KB):
                puts[b][0].wait()
                puts[b][1].wait()
            return carry

        lax.fori_loop(0, nsc, body, 0)

    return gather_k(qc_tab, xc_tab, actr_f, cctr_f, hi, wi)


def _edge_body(g1_r, g2_r, dx_r, dy_r, wd0_r, bd0_r, wd1_r,
               gd_r, bed_r, wc0d_r, gc0_r, bec0_r, h_r):
    wd0 = wd0_r[...]
    d1 = jax.nn.relu(dx_r[...] * wd0[0:1, :]
                     + dy_r[...] * wd0[1:2, :]
                     + bd0_r[...])
    d2 = jax.nn.relu(_gn(jnp.dot(d1, wd1_r[...], preferred_element_type=jnp.float32),
                         gd_r[...], bed_r[...]))
    pre = (jnp.dot(d2, wc0d_r[...], preferred_element_type=jnp.float32)
           + g1_r[...] + g2_r[...])
    h_r[...] = jax.nn.relu(_gn(pre, gc0_r[...], bec0_r[...]))


def _edge_mlp(g1, g2, dx, dy, W_d0, b_d0, W_d1, g_d, be_d,
              Wc0_d, g_c0, be_c0):
    e, d = g1.shape
    blk = 2000
    grid = e // blk
    full = lambda r, c: pl.BlockSpec((r, c), lambda i: (0, 0))
    col = pl.BlockSpec((blk, 1), lambda i: (i, 0))
    return pl.pallas_call(
        _edge_body,
        grid=(grid,),
        in_specs=[
            pl.BlockSpec((blk, d), lambda i: (i, 0)),
            pl.BlockSpec((blk, d), lambda i: (i, 0)),
            col, col,
            full(2, d), full(1, d),
            full(d, d), full(1, d), full(1, d),
            full(d, d), full(1, d), full(1, d),
        ],
        out_specs=pl.BlockSpec((blk, d), lambda i: (i, 0)),
        out_shape=jax.ShapeDtypeStruct((e, d), jnp.float32),
    )(g1, g2, dx, dy, W_d0, b_d0, W_d1, g_d, be_d, Wc0_d, g_c0, be_c0)


_SKB = 2


def _scatter_add(h, hi):
    e, d = h.shape
    per_w = e // _NW
    nch = per_w // _CH
    nsc = nch // _SKB
    stripe = _NPAD // _NS
    zrows = stripe // 10
    mesh = plsc.VectorSubcoreMesh(core_axis_name="c", subcore_axis_name="s",
                                  num_cores=_NC, num_subcores=_NS)

    @functools.partial(
        pl.kernel,
        out_type=jax.ShapeDtypeStruct((_NC, _NPAD, d), jnp.float32),
        mesh=mesh,
        scratch_types=(
            [pltpu.VMEM((per_w,), jnp.int32)]
            + [pltpu.VMEM((_CH, d), jnp.float32)] * _SKB
            + [pltpu.VMEM((zrows, d), jnp.float32)]
            + [pltpu.VMEM_SHARED((_NPAD, d), jnp.float32)]
            + [pltpu.SemaphoreType.DMA] * (2 * _SKB)
        ),
    )
    def scatter_k(h_hbm, hi_hbm, s_out, hi_v, *rest):
        hbuf = rest[:_SKB]
        zbuf = rest[_SKB]
        s_sh = rest[_SKB + 1]
        sr = rest[_SKB + 2:2 * _SKB + 2]
        sa = rest[2 * _SKB + 2:3 * _SKB + 2]
        cid = lax.axis_index("c")
        sid = lax.axis_index("s")
        wid = sid * _NC + cid
        e0 = wid * per_w
        pltpu.sync_copy(hi_hbm.at[pl.ds(e0, per_w)], hi_v)

        def zb(i, carry):
            zbuf[i // 8, pl.ds((i % 8) * 16, 16)] = jnp.zeros((16,), jnp.float32)
            return carry

        lax.fori_loop(0, zrows * 8, zb, 0)

        def zc(p, carry):
            pltpu.sync_copy(zbuf, s_sh.at[pl.ds(sid * stripe + p * zrows, zrows)])
            return carry

        lax.fori_loop(0, 10, zc, 0)
        plsc.subcore_barrier()

        def body(s, carry):
            gets = []
            for b in range(_SKB):
                j = s * _SKB + b
                base = e0 + j * _CH
                gets.append(pltpu.async_copy(
                    h_hbm.at[pl.ds(base, _CH)], hbuf[b], sr[b]))
            adds = []
            for b in range(_SKB):
                j = s * _SKB + b
                gets[b].wait()
                adds.append(pltpu.async_copy(
                    hbuf[b], s_sh.at[hi_v.at[pl.ds(j * _CH, _CH)]], sa[b],
                    add=True))
            for b in range(_SKB):
                adds[b].wait()
            return carry

        lax.fori_loop(0, nsc, body, 0)
        plsc.subcore_barrier()
        pltpu.sync_copy(s_sh.at[pl.ds(sid * stripe, stripe)],
                        s_out.at[cid, pl.ds(sid * stripe, stripe)])

    return scatter_k(h, hi)


def _final_body(s_r, base_r, agts_r, wc1_r, gn_r, ben_r, wl_r, gl_r, bel_r, o_r):
    s = s_r[0] + s_r[1]
    out = base_r[...] + jnp.dot(s, wc1_r[...], preferred_element_type=jnp.float32)
    out = jax.nn.relu(_gn(out, gn_r[...], ben_r[...]))
    out = _gn(jnp.dot(out, wl_r[...], preferred_element_type=jnp.float32),
              gl_r[...], bel_r[...])
    o_r[...] = jax.nn.relu(out + agts_r[...])


def _final(s_parts, base, agts, W_c1, g_n, be_n, W_l, g_l, be_l):
    n, d = agts.shape
    blk = 1000
    grid = n // blk
    full = lambda r, c: pl.BlockSpec((r, c), lambda i: (0, 0))
    return pl.pallas_call(
        _final_body,
        grid=(grid,),
        in_specs=[
            pl.BlockSpec((_NC, blk, d), lambda i: (0, i, 0)),
            pl.BlockSpec((blk, d), lambda i: (i, 0)),
            pl.BlockSpec((blk, d), lambda i: (i, 0)),
            full(d, d), full(1, d), full(1, d),
            full(d, d), full(1, d), full(1, d),
        ],
        out_specs=pl.BlockSpec((blk, d), lambda i: (i, 0)),
        out_shape=jax.ShapeDtypeStruct((n, d), jnp.float32),
    )(s_parts, base, agts, W_c1, g_n, be_n, W_l, g_l, be_l)


def kernel(agts, ctx, agt_ctrs, ctx_ctrs, W_d0, b_d0, W_d1, g_d, be_d,
           W_q, g_q, be_q, W_c0, g_c0, be_c0, W_c1, W_a, g_n, be_n,
           W_l, g_l, be_l, hi, wi):
    n, d = agts.shape
    row = lambda v: v.reshape(1, d)
    Wc0_d, Wc0_q, Wc0_x = W_c0[:d], W_c0[d:2 * d], W_c0[2 * d:]

    qc_tab, xc_tab, base = _build_tables(
        agts, ctx, W_q, row(g_q), row(be_q), Wc0_q, Wc0_x, W_a)

    g1, g2, dx, dy = _gather_rows(qc_tab, xc_tab, agt_ctrs.reshape(-1),
                                  ctx_ctrs.reshape(-1), hi, wi)

    e = hi.shape[0]
    h = _edge_mlp(g1, g2, dx.reshape(e, 1), dy.reshape(e, 1), W_d0,
                  row(b_d0), W_d1, row(g_d), row(be_d), Wc0_d,
                  row(g_c0), row(be_c0))

    s_parts = _scatter_add(h, hi)[:, :n, :]

    return _final(s_parts, base, agts, W_c1, row(g_n), row(be_n), W_l,
                  row(g_l), row(be_l))


# speedup vs baseline: 1.4868x; 1.0276x over previous
import functools

import jax
import jax.numpy as jnp
from jax import lax
from jax.experimental import pallas as pl
from jax.experimental.pallas import tpu as pltpu
from jax.experimental.pallas import tpu_sc as plsc

_NC = 2
_NS = 16
_NW = _NC * _NS
_CH = 40
_KB = 5
_NPAD = 10240

_EPS = 1e-5


def _gn(x, g, b):
    mu = jnp.mean(x, axis=1, keepdims=True)
    var = jnp.mean((x - mu) ** 2, axis=1, keepdims=True)
    return (x - mu) / jnp.sqrt(var + _EPS) * g + b


def _pre_body(agts_r, ctx_r, wq_r, gq_r, beq_r, wc0q_r, wc0x_r, wa_r,
              qc_r, xc_r, base_r):
    agts = agts_r[...]
    q = jax.nn.relu(_gn(jnp.dot(agts, wq_r[...], preferred_element_type=jnp.float32),
                        gq_r[...], beq_r[...]))
    qc_r[...] = jnp.dot(q, wc0q_r[...], preferred_element_type=jnp.float32)
    xc_r[...] = jnp.dot(ctx_r[...], wc0x_r[...], preferred_element_type=jnp.float32)
    base_r[...] = jnp.dot(agts, wa_r[...], preferred_element_type=jnp.float32)


def _build_tables(agts, ctx, W_q, g_q, be_q, Wc0_q, Wc0_x, W_a):
    n, d = agts.shape
    blk = 1000
    grid = n // blk
    full = lambda r, c: pl.BlockSpec((r, c), lambda i: (0, 0))
    return pl.pallas_call(
        _pre_body,
        grid=(grid,),
        in_specs=[
            pl.BlockSpec((blk, d), lambda i: (i, 0)),
            pl.BlockSpec((blk, d), lambda i: (i, 0)),
            full(d, d), full(1, d), full(1, d),
            full(d, d), full(d, d), full(d, d),
        ],
        out_specs=[
            pl.BlockSpec((blk, d), lambda i: (i, 0)),
            pl.BlockSpec((blk, d), lambda i: (i, 0)),
            pl.BlockSpec((blk, d), lambda i: (i, 0)),
        ],
        out_shape=[
            jax.ShapeDtypeStruct((n, d), jnp.float32),
            jax.ShapeDtypeStruct((n, d), jnp.float32),
            jax.ShapeDtypeStruct((n, d), jnp.float32),
        ],
    )(agts, ctx, W_q, g_q, be_q, Wc0_q, Wc0_x, W_a)


def _gather_rows(qc_tab, xc_tab, actr_f, cctr_f, hi, wi):
    e = hi.shape[0]
    n, d = qc_tab.shape
    n2 = actr_f.shape[0]
    per_w = e // _NW
    nch = per_w // _CH
    nsc = nch // _KB
    nv = per_w // 16
    mesh = plsc.VectorSubcoreMesh(core_axis_name="c", subcore_axis_name="s",
                                  num_cores=_NC, num_subcores=_NS)

    @functools.partial(
        pl.kernel,
        out_type=(jax.ShapeDtypeStruct((e, d), jnp.float32),
                  jax.ShapeDtypeStruct((e,), jnp.float32),
                  jax.ShapeDtypeStruct((e,), jnp.float32)),
        mesh=mesh,
        compiler_params=pltpu.CompilerParams(needs_layout_passes=False),
        scratch_types=(
            [pltpu.VMEM((per_w,), jnp.int32)] * 2
            + [pltpu.VMEM((n2,), jnp.float32)] * 2
            + [pltpu.VMEM((per_w,), jnp.float32)]
            + [pltpu.VMEM((_CH, 128), jnp.float32)] * (2 * _KB)
            + [pltpu.SemaphoreType.DMA] * (2 * _KB + 1)
        ),
    )
    def gather_k(qc_hbm, xc_hbm, actr_hbm, cctr_hbm,
                 hi_hbm, wi_hbm,
                 g_hbm, dx_o, dy_o,
                 hi_v, wi_v, actr_v, cctr_v, dbuf, *bufs_and_sems):
        r1 = bufs_and_sems[:_KB]
        r2 = bufs_and_sems[_KB:2 * _KB]
        sg = bufs_and_sems[2 * _KB:3 * _KB]
        sw = bufs_and_sems[3 * _KB:4 * _KB]
        s3 = bufs_and_sems[4 * _KB]
        wid = lax.axis_index("s") * _NC + lax.axis_index("c")
        e0 = wid * per_w
        pltpu.sync_copy(hi_hbm.at[pl.ds(e0, per_w)], hi_v)
        pltpu.sync_copy(wi_hbm.at[pl.ds(e0, per_w)], wi_v)
        pltpu.sync_copy(actr_hbm, actr_v)
        pltpu.sync_copy(cctr_hbm, cctr_v)

        def diff(off, out_ref):
            def dbody(i, c):
                sl = pl.ds(i * 16, 16)
                a = plsc.load_gather(actr_v, [hi_v[sl] * 2 + off])
                cc = plsc.load_gather(cctr_v, [wi_v[sl] * 2 + off])
                dbuf[sl] = a - cc
                return c

            lax.fori_loop(0, nv, dbody, 0)
            pltpu.sync_copy(dbuf, out_ref.at[pl.ds(e0, per_w)])

        diff(0, dx_o)
        diff(1, dy_o)

        def body(s, carry):
            gets = []
            for b in range(_KB):
                j = s * _KB + b
                ids = pl.ds(j * _CH, _CH)
                gets.append((
                    pltpu.async_copy(qc_hbm.at[hi_v.at[ids]], r1[b], sg[b]),
                    pltpu.async_copy(xc_hbm.at[wi_v.at[ids]], r2[b], sg[b]),
                ))
            puts = []
            for b in range(_KB):
                j = s * _KB + b
                base = e0 + j * _CH
                gets[b][0].wait()
                gets[b][1].wait()
                r1b, r2b = r1[b], r2[b]

                def addrow(i, c, r1b=r1b, r2b=r2b):
                    for q in range(8):
                        sl = pl.ds(q * 16, 16)
                        r1b[i, sl] = r1b[i, sl] + r2b[i, sl]
                    return c

                lax.fori_loop(0, _CH, addrow, 0)
                puts.append(
                    pltpu.async_copy(r1b, g_hbm.at[pl.ds(base, _CH)], sw[b]))
            for b in range(_KB):
                puts[b].wait()
            return carry

        lax.fori_loop(0, nsc, body, 0)

    return gather_k(qc_tab, xc_tab, actr_f, cctr_f, hi, wi)


def _edge_body(g_r, dx_r, dy_r, wd0_r, bd0_r, wd1_r,
               gd_r, bed_r, wc0d_r, gc0_r, bec0_r, h_r):
    wd0 = wd0_r[...]
    d1 = jax.nn.relu(dx_r[...] * wd0[0:1, :]
                     + dy_r[...] * wd0[1:2, :]
                     + bd0_r[...])
    d2 = jax.nn.relu(_gn(jnp.dot(d1, wd1_r[...], preferred_element_type=jnp.float32),
                         gd_r[...], bed_r[...]))
    pre = (jnp.dot(d2, wc0d_r[...], preferred_element_type=jnp.float32)
           + g_r[...])
    h_r[...] = jax.nn.relu(_gn(pre, gc0_r[...], bec0_r[...]))


def _edge_mlp(g, dx, dy, W_d0, b_d0, W_d1, g_d, be_d,
              Wc0_d, g_c0, be_c0):
    e, d = g.shape
    blk = 2000
    grid = e // blk
    full = lambda r, c: pl.BlockSpec((r, c), lambda i: (0, 0))
    col = pl.BlockSpec((blk, 1), lambda i: (i, 0))
    return pl.pallas_call(
        _edge_body,
        grid=(grid,),
        in_specs=[
            pl.BlockSpec((blk, d), lambda i: (i, 0)),
            col, col,
            full(2, d), full(1, d),
            full(d, d), full(1, d), full(1, d),
            full(d, d), full(1, d), full(1, d),
        ],
        out_specs=pl.BlockSpec((blk, d), lambda i: (i, 0)),
        out_shape=jax.ShapeDtypeStruct((e, d), jnp.float32),
    )(g, dx, dy, W_d0, b_d0, W_d1, g_d, be_d, Wc0_d, g_c0, be_c0)


_SKB = 2


def _scatter_add(h, hi):
    e, d = h.shape
    per_w = e // _NW
    nch = per_w // _CH
    nsc = nch // _SKB
    stripe = _NPAD // _NS
    zrows = stripe // 10
    mesh = plsc.VectorSubcoreMesh(core_axis_name="c", subcore_axis_name="s",
                                  num_cores=_NC, num_subcores=_NS)

    @functools.partial(
        pl.kernel,
        out_type=jax.ShapeDtypeStruct((_NC, _NPAD, d), jnp.float32),
        mesh=mesh,
        scratch_types=(
            [pltpu.VMEM((per_w,), jnp.int32)]
            + [pltpu.VMEM((_CH, d), jnp.float32)] * _SKB
            + [pltpu.VMEM((zrows, d), jnp.float32)]
            + [pltpu.VMEM_SHARED((_NPAD, d), jnp.float32)]
            + [pltpu.SemaphoreType.DMA] * (2 * _SKB)
        ),
    )
    def scatter_k(h_hbm, hi_hbm, s_out, hi_v, *rest):
        hbuf = rest[:_SKB]
        zbuf = rest[_SKB]
        s_sh = rest[_SKB + 1]
        sr = rest[_SKB + 2:2 * _SKB + 2]
        sa = rest[2 * _SKB + 2:3 * _SKB + 2]
        cid = lax.axis_index("c")
        sid = lax.axis_index("s")
        wid = sid * _NC + cid
        e0 = wid * per_w
        pltpu.sync_copy(hi_hbm.at[pl.ds(e0, per_w)], hi_v)

        def zb(i, carry):
            zbuf[i // 8, pl.ds((i % 8) * 16, 16)] = jnp.zeros((16,), jnp.float32)
            return carry

        lax.fori_loop(0, zrows * 8, zb, 0)

        def zc(p, carry):
            pltpu.sync_copy(zbuf, s_sh.at[pl.ds(sid * stripe + p * zrows, zrows)])
            return carry

        lax.fori_loop(0, 10, zc, 0)
        plsc.subcore_barrier()

        def body(s, carry):
            gets = []
            for b in range(_SKB):
                j = s * _SKB + b
                base = e0 + j * _CH
                gets.append(pltpu.async_copy(
                    h_hbm.at[pl.ds(base, _CH)], hbuf[b], sr[b]))
            adds = []
            for b in range(_SKB):
                j = s * _SKB + b
                gets[b].wait()
                adds.append(pltpu.async_copy(
                    hbuf[b], s_sh.at[hi_v.at[pl.ds(j * _CH, _CH)]], sa[b],
                    add=True))
            for b in range(_SKB):
                adds[b].wait()
            return carry

        lax.fori_loop(0, nsc, body, 0)
        plsc.subcore_barrier()
        pltpu.sync_copy(s_sh.at[pl.ds(sid * stripe, stripe)],
                        s_out.at[cid, pl.ds(sid * stripe, stripe)])

    return scatter_k(h, hi)


def _final_body(s_r, base_r, agts_r, wc1_r, gn_r, ben_r, wl_r, gl_r, bel_r, o_r):
    s = s_r[0] + s_r[1]
    out = base_r[...] + jnp.dot(s, wc1_r[...], preferred_element_type=jnp.float32)
    out = jax.nn.relu(_gn(out, gn_r[...], ben_r[...]))
    out = _gn(jnp.dot(out, wl_r[...], preferred_element_type=jnp.float32),
              gl_r[...], bel_r[...])
    o_r[...] = jax.nn.relu(out + agts_r[...])


def _final(s_parts, base, agts, W_c1, g_n, be_n, W_l, g_l, be_l):
    n, d = agts.shape
    blk = 1000
    grid = n // blk
    full = lambda r, c: pl.BlockSpec((r, c), lambda i: (0, 0))
    return pl.pallas_call(
        _final_body,
        grid=(grid,),
        in_specs=[
            pl.BlockSpec((_NC, blk, d), lambda i: (0, i, 0)),
            pl.BlockSpec((blk, d), lambda i: (i, 0)),
            pl.BlockSpec((blk, d), lambda i: (i, 0)),
            full(d, d), full(1, d), full(1, d),
            full(d, d), full(1, d), full(1, d),
        ],
        out_specs=pl.BlockSpec((blk, d), lambda i: (i, 0)),
        out_shape=jax.ShapeDtypeStruct((n, d), jnp.float32),
    )(s_parts, base, agts, W_c1, g_n, be_n, W_l, g_l, be_l)


def kernel(agts, ctx, agt_ctrs, ctx_ctrs, W_d0, b_d0, W_d1, g_d, be_d,
           W_q, g_q, be_q, W_c0, g_c0, be_c0, W_c1, W_a, g_n, be_n,
           W_l, g_l, be_l, hi, wi):
    n, d = agts.shape
    row = lambda v: v.reshape(1, d)
    Wc0_d, Wc0_q, Wc0_x = W_c0[:d], W_c0[d:2 * d], W_c0[2 * d:]

    qc_tab, xc_tab, base = _build_tables(
        agts, ctx, W_q, row(g_q), row(be_q), Wc0_q, Wc0_x, W_a)

    g, dx, dy = _gather_rows(qc_tab, xc_tab, agt_ctrs.reshape(-1),
                             ctx_ctrs.reshape(-1), hi, wi)

    e = hi.shape[0]
    h = _edge_mlp(g, dx.reshape(e, 1), dy.reshape(e, 1), W_d0,
                  row(b_d0), W_d1, row(g_d), row(be_d), Wc0_d,
                  row(g_c0), row(be_c0))

    s_parts = _scatter_add(h, hi)[:, :n, :]

    return _final(s_parts, base, agts, W_c1, row(g_n), row(be_n), W_l,
                  row(g_l), row(be_l))
